# baseline (speedup 1.0000x reference)
import functools

import jax
import jax.numpy as jnp
from jax import lax
from jax.experimental import pallas as pl
from jax.experimental.pallas import tpu as pltpu
from jax.experimental.pallas import tpu_sc as plsc

B, S, D_MODEL = 16, 512, 256
MAX_T = 2048
F = 256

_T_TILE = MAX_T // 2
_GCH = 128


def _expand_sc_body(h_hbm, dgt_hbm, out_hbm,
                    d_v, c_v, cnt_v, idx_v, rows_v, sem0, sem1):
    i32 = jnp.int32
    wid = lax.axis_index("s") * 2 + lax.axis_index("c")
    b = wid // 2
    t0 = (wid % 2) * _T_TILE

    pltpu.sync_copy(dgt_hbm.at[b], d_v)

    carry = jnp.zeros((), i32)
    for j in range(S // 16):
        x = jnp.maximum(d_v[pl.ds(j * 16, 16)], 0)
        c_v[pl.ds(j * 16, 16)] = plsc.cumsum(x) + carry
        carry = carry + jnp.sum(x)
    total = carry

    zeros16 = jnp.zeros((16,), i32)
    for k in range(_T_TILE // 16):
        cnt_v[pl.ds(k * 16, 16)] = zeros16
    base = jnp.zeros((), i32)
    ones16 = jnp.full((16,), 1, i32)
    for j in range(S // 16):
        cj = c_v[pl.ds(j * 16, 16)]
        pos = cj - t0
        in_tile = (pos >= 0) & (pos < _T_TILE)
        plsc.addupdate_scatter(cnt_v, [pos], ones16, mask=in_tile)
        base = base + jnp.sum(jnp.where(cj < t0, 1, 0).astype(i32))

    run = base
    boff = b * S
    for k in range(_T_TILE // 16):
        cum = plsc.cumsum(cnt_v[pl.ds(k * 16, 16)]) + run
        run = run + jnp.sum(cnt_v[pl.ds(k * 16, 16)])
        idx_v[pl.ds(k * 16, 16)] = jnp.where(cum < S, cum + boff, boff)

    out_base = b * MAX_T + t0
    zf = jnp.zeros((16,), jnp.float32)

    def _zero_tail(buf, k):
        nv = jnp.clip(total - (t0 + k * _GCH), 0, _GCH)

        def body(r, _):
            for w in range(D_MODEL // 16):
                buf[r, pl.ds(w * 16, 16)] = zf
            return 0

        lax.fori_loop(nv, _GCH, body, 0)

    sems = (sem0, sem1)
    nch = _T_TILE // _GCH
    cps = [None] * nch
    for k in range(nch):
        cps[k] = pltpu.async_copy(
            h_hbm.at[idx_v.at[pl.ds(k * _GCH, _GCH)]], rows_v.at[k % 2],
            sems[k % 2])
        if k > 0:
            cps[k - 1].wait()
            _zero_tail(rows_v.at[(k - 1) % 2], k - 1)
            pltpu.sync_copy(rows_v.at[(k - 1) % 2],
                            out_hbm.at[pl.ds(out_base + (k - 1) * _GCH, _GCH)])
    cps[nch - 1].wait()
    _zero_tail(rows_v.at[(nch - 1) % 2], nch - 1)
    pltpu.sync_copy(rows_v.at[(nch - 1) % 2],
                    out_hbm.at[pl.ds(out_base + (nch - 1) * _GCH, _GCH)])


def _conv3(x, w_ref, b):
    cin = x.shape[1]
    xc = x if x.dtype == jnp.bfloat16 else x.astype(jnp.bfloat16)
    zrow = jnp.zeros((1, cin), jnp.bfloat16)
    xm = jnp.concatenate([zrow, xc[:-1]], axis=0)
    xp = jnp.concatenate([xc[1:], zrow], axis=0)
    y = jnp.dot(xm, w_ref[0], preferred_element_type=jnp.float32)
    y = y + jnp.dot(xc, w_ref[1], preferred_element_type=jnp.float32)
    y = y + jnp.dot(xp, w_ref[2], preferred_element_type=jnp.float32)
    return y + b


def _fused_body(hexp_ref, h_ref, pgt_ref, egt_ref,
                dw1_ref, db1_ref, dw2_ref, db2_ref, dwl_ref, dbl_ref,
                wj1_ref, bj1_ref, pw2_ref, pb2_ref, ew2_ref, eb2_ref,
                pwl_ref, pbl_ref, ewl_ref, ebl_ref,
                ppjw_ref, epjw_ref, pjb_ref,
                ha_ref, dp_ref, pp_ref, ep_ref):
    xd = h_ref[0]
    g1 = jax.nn.relu(_conv3(xd, dw1_ref, db1_ref[...]))
    g2 = jax.nn.relu(_conv3(g1, dw2_ref, db2_ref[...]))
    dp_ref[0] = jnp.dot(g2, dwl_ref[...], preferred_element_type=jnp.float32) + dbl_ref[...]

    x = hexp_ref[0]
    p = pgt_ref[0]
    e = egt_ref[0]
    ha_ref[0] = x + (p * ppjw_ref[...] + (e * epjw_ref[...] + pjb_ref[...]))
    h1 = jax.nn.relu(_conv3(x, wj1_ref, bj1_ref[...]))
    h2p = jax.nn.relu(_conv3(h1[:, :F], pw2_ref, pb2_ref[...]))
    h2e = jax.nn.relu(_conv3(h1[:, F:], ew2_ref, eb2_ref[...]))
    pp_ref[0] = jnp.dot(h2p, pwl_ref[...], preferred_element_type=jnp.float32) + pbl_ref[...]
    ep_ref[0] = jnp.dot(h2e, ewl_ref[...], preferred_element_type=jnp.float32) + ebl_ref[...]


def _full(bs):
    n = len(bs)
    return pl.BlockSpec(bs, lambda b: (b,) + (0,) * (n - 1))


def _rep(bs):
    n = len(bs)
    return pl.BlockSpec(bs, lambda b: (0,) * n)


def kernel(H, D_gt, P_gt, E_gt, dp_w1, dp_b1, dp_w2, dp_b2, dp_wl, dp_bl,
           pp_w1, pp_b1, pp_w2, pp_b2, pp_wl, pp_bl,
           ep_w1, ep_b1, ep_w2, ep_b2, ep_wl, ep_bl,
           ppj_w, ppj_b, epj_w, epj_b):
    f32 = jnp.float32
    bf16 = jnp.bfloat16
    def taps(w):
        return jnp.transpose(w, (2, 1, 0)).astype(bf16)
    dp_w1t, dp_w2t = taps(dp_w1), taps(dp_w2)
    wj1 = jnp.concatenate([taps(pp_w1), taps(ep_w1)], axis=2)
    bj1 = jnp.concatenate([pp_b1, ep_b1])[None, :]
    pw2t, ew2t = taps(pp_w2), taps(ep_w2)

    expand = pl.kernel(
        _expand_sc_body,
        out_type=jax.ShapeDtypeStruct((B * MAX_T, D_MODEL), f32),
        mesh=plsc.VectorSubcoreMesh(core_axis_name="c", subcore_axis_name="s"),
        scratch_types=[
            pltpu.VMEM((S,), jnp.int32),
            pltpu.VMEM((S,), jnp.int32),
            pltpu.VMEM((_T_TILE,), jnp.int32),
            pltpu.VMEM((_T_TILE,), jnp.int32),
            pltpu.VMEM((2, _GCH, D_MODEL), f32),
            pltpu.SemaphoreType.DMA,
            pltpu.SemaphoreType.DMA,
        ],
        compiler_params=pltpu.CompilerParams(needs_layout_passes=False),
    )
    h_exp = expand(H.reshape(B * S, D_MODEL), D_gt).reshape(B, MAX_T, D_MODEL)

    ha, d_pred, ppred, epred = pl.pallas_call(
        _fused_body,
        grid=(B,),
        in_specs=[_full((1, MAX_T, D_MODEL)),
                  _full((1, S, D_MODEL)),
                  _full((1, MAX_T, 1)), _full((1, MAX_T, 1)),
                  _rep((3, D_MODEL, F)), _rep((1, F)),
                  _rep((3, F, F)), _rep((1, F)), _rep((F, 1)), _rep((1, 1)),
                  _rep((3, D_MODEL, 2 * F)), _rep((1, 2 * F)),
                  _rep((3, F, F)), _rep((1, F)), _rep((3, F, F)), _rep((1, F)),
                  _rep((F, 1)), _rep((1, 1)), _rep((F, 1)), _rep((1, 1)),
                  _rep((1, D_MODEL)), _rep((1, D_MODEL)), _rep((1, D_MODEL))],
        out_specs=[_full((1, MAX_T, D_MODEL)), _full((1, S, 1)),
                   _full((1, MAX_T, 1)), _full((1, MAX_T, 1))],
        out_shape=[jax.ShapeDtypeStruct((B, MAX_T, D_MODEL), f32),
                   jax.ShapeDtypeStruct((B, S, 1), f32),
                   jax.ShapeDtypeStruct((B, MAX_T, 1), f32),
                   jax.ShapeDtypeStruct((B, MAX_T, 1), f32)],
    )(h_exp, H, P_gt[..., None], E_gt[..., None],
      dp_w1t, dp_b1[None, :], dp_w2t, dp_b2[None, :], dp_wl, dp_bl[None, :],
      wj1, bj1, pw2t, pp_b2[None, :], ew2t, ep_b2[None, :],
      pp_wl, pp_bl[None, :], ep_wl, ep_bl[None, :],
      ppj_w[None, :], epj_w[None, :], (ppj_b + epj_b)[None, :])

    return (ha, d_pred[..., 0], ppred[..., 0], epred[..., 0])


# speedup vs baseline: 1.0039x; 1.0039x over previous
import functools

import jax
import jax.numpy as jnp
from jax import lax
from jax.experimental import pallas as pl
from jax.experimental.pallas import tpu as pltpu
from jax.experimental.pallas import tpu_sc as plsc

B, S, D_MODEL = 16, 512, 256
MAX_T = 2048
F = 256

_T_TILE = MAX_T // 2
_GCH = 128


def _expand_sc_body(h_hbm, dgt_hbm, out_hbm,
                    d_v, c_v, cnt_v, idx_v, rows_v, sem0, sem1, sem2):
    i32 = jnp.int32
    wid = lax.axis_index("s") * 2 + lax.axis_index("c")
    b = wid // 2
    t0 = (wid % 2) * _T_TILE

    pltpu.sync_copy(dgt_hbm.at[b], d_v)

    carry = jnp.zeros((), i32)
    for j in range(S // 16):
        x = jnp.maximum(d_v[pl.ds(j * 16, 16)], 0)
        c_v[pl.ds(j * 16, 16)] = plsc.cumsum(x) + carry
        carry = carry + jnp.sum(x)
    total = carry

    zeros16 = jnp.zeros((16,), i32)
    for k in range(_T_TILE // 16):
        cnt_v[pl.ds(k * 16, 16)] = zeros16
    base = jnp.zeros((), i32)
    ones16 = jnp.full((16,), 1, i32)
    for j in range(S // 16):
        cj = c_v[pl.ds(j * 16, 16)]
        pos = cj - t0
        in_tile = (pos >= 0) & (pos < _T_TILE)
        plsc.addupdate_scatter(cnt_v, [pos], ones16, mask=in_tile)
        base = base + jnp.sum(jnp.where(cj < t0, 1, 0).astype(i32))

    run = base
    boff = b * S
    for k in range(_T_TILE // 16):
        cum = plsc.cumsum(cnt_v[pl.ds(k * 16, 16)]) + run
        run = run + jnp.sum(cnt_v[pl.ds(k * 16, 16)])
        idx_v[pl.ds(k * 16, 16)] = jnp.where(cum < S, cum + boff, boff)

    out_base = b * MAX_T + t0
    zf = jnp.zeros((16,), jnp.float32)

    def _zero_tail(buf, k):
        nv = jnp.clip(total - (t0 + k * _GCH), 0, _GCH)

        def body(r, _):
            for w in range(D_MODEL // 16):
                buf[r, pl.ds(w * 16, 16)] = zf
            return 0

        lax.fori_loop(nv, _GCH, body, 0)

    sems = (sem0, sem1, sem2)
    nbuf = 3
    nch = _T_TILE // _GCH
    cps = [None] * nch

    def _drain(k):
        cps[k].wait()
        _zero_tail(rows_v.at[k % nbuf], k)
        pltpu.sync_copy(rows_v.at[k % nbuf],
                        out_hbm.at[pl.ds(out_base + k * _GCH, _GCH)])

    for k in range(nch):
        cps[k] = pltpu.async_copy(
            h_hbm.at[idx_v.at[pl.ds(k * _GCH, _GCH)]], rows_v.at[k % nbuf],
            sems[k % nbuf])
        if k >= nbuf - 1:
            _drain(k - (nbuf - 1))
    for k in range(nch - (nbuf - 1), nch):
        _drain(k)


def _conv3(x, w_ref, b):
    cin = x.shape[1]
    xc = x if x.dtype == jnp.bfloat16 else x.astype(jnp.bfloat16)
    zrow = jnp.zeros((1, cin), jnp.bfloat16)
    xm = jnp.concatenate([zrow, xc[:-1]], axis=0)
    xp = jnp.concatenate([xc[1:], zrow], axis=0)
    y = jnp.dot(xm, w_ref[0], preferred_element_type=jnp.float32)
    y = y + jnp.dot(xc, w_ref[1], preferred_element_type=jnp.float32)
    y = y + jnp.dot(xp, w_ref[2], preferred_element_type=jnp.float32)
    return y + b


def _fused_body(hexp_ref, h_ref, pgt_ref, egt_ref,
                dw1_ref, db1_ref, dw2_ref, db2_ref, dwl_ref, dbl_ref,
                wj1_ref, bj1_ref, pw2_ref, pb2_ref, ew2_ref, eb2_ref,
                pwl_ref, pbl_ref, ewl_ref, ebl_ref,
                ppjw_ref, epjw_ref, pjb_ref,
                ha_ref, dp_ref, pp_ref, ep_ref):
    xd = h_ref[0]
    g1 = jax.nn.relu(_conv3(xd, dw1_ref, db1_ref[...]))
    g2 = jax.nn.relu(_conv3(g1, dw2_ref, db2_ref[...]))
    dp_ref[0] = jnp.dot(g2, dwl_ref[...], preferred_element_type=jnp.float32) + dbl_ref[...]

    x = hexp_ref[0]
    p = pgt_ref[0]
    e = egt_ref[0]
    ha_ref[0] = x + (p * ppjw_ref[...] + (e * epjw_ref[...] + pjb_ref[...]))
    h1 = jax.nn.relu(_conv3(x, wj1_ref, bj1_ref[...]))
    h2p = jax.nn.relu(_conv3(h1[:, :F], pw2_ref, pb2_ref[...]))
    h2e = jax.nn.relu(_conv3(h1[:, F:], ew2_ref, eb2_ref[...]))
    pp_ref[0] = jnp.dot(h2p, pwl_ref[...], preferred_element_type=jnp.float32) + pbl_ref[...]
    ep_ref[0] = jnp.dot(h2e, ewl_ref[...], preferred_element_type=jnp.float32) + ebl_ref[...]


def _full(bs):
    n = len(bs)
    return pl.BlockSpec(bs, lambda b: (b,) + (0,) * (n - 1))


def _rep(bs):
    n = len(bs)
    return pl.BlockSpec(bs, lambda b: (0,) * n)


def kernel(H, D_gt, P_gt, E_gt, dp_w1, dp_b1, dp_w2, dp_b2, dp_wl, dp_bl,
           pp_w1, pp_b1, pp_w2, pp_b2, pp_wl, pp_bl,
           ep_w1, ep_b1, ep_w2, ep_b2, ep_wl, ep_bl,
           ppj_w, ppj_b, epj_w, epj_b):
    f32 = jnp.float32
    bf16 = jnp.bfloat16
    def taps(w):
        return jnp.transpose(w, (2, 1, 0)).astype(bf16)
    dp_w1t, dp_w2t = taps(dp_w1), taps(dp_w2)
    wj1 = jnp.concatenate([taps(pp_w1), taps(ep_w1)], axis=2)
    bj1 = jnp.concatenate([pp_b1, ep_b1])[None, :]
    pw2t, ew2t = taps(pp_w2), taps(ep_w2)

    expand = pl.kernel(
        _expand_sc_body,
        out_type=jax.ShapeDtypeStruct((B * MAX_T, D_MODEL), f32),
        mesh=plsc.VectorSubcoreMesh(core_axis_name="c", subcore_axis_name="s"),
        scratch_types=[
            pltpu.VMEM((S,), jnp.int32),
            pltpu.VMEM((S,), jnp.int32),
            pltpu.VMEM((_T_TILE,), jnp.int32),
            pltpu.VMEM((_T_TILE,), jnp.int32),
            pltpu.VMEM((3, _GCH, D_MODEL), f32),
            pltpu.SemaphoreType.DMA,
            pltpu.SemaphoreType.DMA,
            pltpu.SemaphoreType.DMA,
        ],
        compiler_params=pltpu.CompilerParams(needs_layout_passes=False),
    )
    h_exp = expand(H.reshape(B * S, D_MODEL), D_gt).reshape(B, MAX_T, D_MODEL)

    ha, d_pred, ppred, epred = pl.pallas_call(
        _fused_body,
        grid=(B,),
        in_specs=[_full((1, MAX_T, D_MODEL)),
                  _full((1, S, D_MODEL)),
                  _full((1, MAX_T, 1)), _full((1, MAX_T, 1)),
                  _rep((3, D_MODEL, F)), _rep((1, F)),
                  _rep((3, F, F)), _rep((1, F)), _rep((F, 1)), _rep((1, 1)),
                  _rep((3, D_MODEL, 2 * F)), _rep((1, 2 * F)),
                  _rep((3, F, F)), _rep((1, F)), _rep((3, F, F)), _rep((1, F)),
                  _rep((F, 1)), _rep((1, 1)), _rep((F, 1)), _rep((1, 1)),
                  _rep((1, D_MODEL)), _rep((1, D_MODEL)), _rep((1, D_MODEL))],
        out_specs=[_full((1, MAX_T, D_MODEL)), _full((1, S, 1)),
                   _full((1, MAX_T, 1)), _full((1, MAX_T, 1))],
        out_shape=[jax.ShapeDtypeStruct((B, MAX_T, D_MODEL), f32),
                   jax.ShapeDtypeStruct((B, S, 1), f32),
                   jax.ShapeDtypeStruct((B, MAX_T, 1), f32),
                   jax.ShapeDtypeStruct((B, MAX_T, 1), f32)],
    )(h_exp, H, P_gt[..., None], E_gt[..., None],
      dp_w1t, dp_b1[None, :], dp_w2t, dp_b2[None, :], dp_wl, dp_bl[None, :],
      wj1, bj1, pw2t, pp_b2[None, :], ew2t, ep_b2[None, :],
      pp_wl, pp_bl[None, :], ep_wl, ep_bl[None, :],
      ppj_w[None, :], epj_w[None, :], (ppj_b + epj_b)[None, :])

    return (ha, d_pred[..., 0], ppred[..., 0], epred[..., 0])


# speedup vs baseline: 1.0141x; 1.0102x over previous
import functools

import jax
import jax.numpy as jnp
from jax import lax
from jax.experimental import pallas as pl
from jax.experimental.pallas import tpu as pltpu
from jax.experimental.pallas import tpu_sc as plsc

B, S, D_MODEL = 16, 512, 256
MAX_T = 2048
F = 256
_HW = D_MODEL // 2

_T_TILE = MAX_T // 2
_GCH = 128


def _expand_sc_body(h_hbm, dgt_hbm, out_hbm,
                    d_v, c_v, cnt_v, idx_v, rows_v, sem0, sem1, sem2):
    i32 = jnp.int32
    wid = lax.axis_index("s") * 2 + lax.axis_index("c")
    b = wid // 2
    t0 = (wid % 2) * _T_TILE

    pltpu.sync_copy(dgt_hbm.at[b], d_v)

    carry = jnp.zeros((), i32)
    for j in range(S // 16):
        x = jnp.maximum(d_v[pl.ds(j * 16, 16)], 0)
        c_v[pl.ds(j * 16, 16)] = plsc.cumsum(x) + carry
        carry = carry + jnp.sum(x)
    total = carry

    zeros16 = jnp.zeros((16,), i32)
    for k in range(_T_TILE // 16):
        cnt_v[pl.ds(k * 16, 16)] = zeros16
    base = jnp.zeros((), i32)
    ones16 = jnp.full((16,), 1, i32)
    for j in range(S // 16):
        cj = c_v[pl.ds(j * 16, 16)]
        pos = cj - t0
        in_tile = (pos >= 0) & (pos < _T_TILE)
        plsc.addupdate_scatter(cnt_v, [pos], ones16, mask=in_tile)
        base = base + jnp.sum(jnp.where(cj < t0, 1, 0).astype(i32))

    run = base
    boff = b * S
    for k in range(_T_TILE // 16):
        cum = plsc.cumsum(cnt_v[pl.ds(k * 16, 16)]) + run
        run = run + jnp.sum(cnt_v[pl.ds(k * 16, 16)])
        idx_v[pl.ds(k * 16, 16)] = jnp.where(cum < S, cum + boff, boff)

    out_base = b * MAX_T + t0
    zf = jnp.zeros((16,), i32)

    def _zero_tail(buf, k):
        nv = jnp.clip(total - (t0 + k * _GCH), 0, _GCH)

        def body(r, _):
            for w in range(_HW // 16):
                buf[r, pl.ds(w * 16, 16)] = zf
            return 0

        lax.fori_loop(nv, _GCH, body, 0)

    sems = (sem0, sem1, sem2)
    nbuf = 3
    nch = _T_TILE // _GCH
    cps = [None] * nch

    def _drain(k):
        cps[k].wait()
        _zero_tail(rows_v.at[k % nbuf], k)
        pltpu.sync_copy(rows_v.at[k % nbuf],
                        out_hbm.at[pl.ds(out_base + k * _GCH, _GCH)])

    for k in range(nch):
        cps[k] = pltpu.async_copy(
            h_hbm.at[idx_v.at[pl.ds(k * _GCH, _GCH)]], rows_v.at[k % nbuf],
            sems[k % nbuf])
        if k >= nbuf - 1:
            _drain(k - (nbuf - 1))
    for k in range(nch - (nbuf - 1), nch):
        _drain(k)


def _conv3(x, w_ref, b):
    cin = x.shape[1]
    xc = x if x.dtype == jnp.bfloat16 else x.astype(jnp.bfloat16)
    zrow = jnp.zeros((1, cin), jnp.bfloat16)
    xm = jnp.concatenate([zrow, xc[:-1]], axis=0)
    xp = jnp.concatenate([xc[1:], zrow], axis=0)
    y = jnp.dot(xm, w_ref[0], preferred_element_type=jnp.float32)
    y = y + jnp.dot(xc, w_ref[1], preferred_element_type=jnp.float32)
    y = y + jnp.dot(xp, w_ref[2], preferred_element_type=jnp.float32)
    return y + b


_ROUND = 0x8000
_HIMASK = -65536


def _dp_pack_body(h_ref, w1_ref, b1_ref, w2_ref, b2_ref, wl_ref, bl_ref,
                  dp_ref, hw_ref):
    x = h_ref[0]
    g1 = jax.nn.relu(_conv3(x, w1_ref, b1_ref[...]))
    g2 = jax.nn.relu(_conv3(g1, w2_ref, b2_ref[...]))
    dp_ref[0] = jnp.dot(g2, wl_ref[...], preferred_element_type=jnp.float32) + bl_ref[...]
    bits = lax.bitcast_convert_type(x, jnp.int32)
    lo = lax.shift_right_logical(bits[:, :_HW] + _ROUND, 16)
    hi = (bits[:, _HW:] + _ROUND) & _HIMASK
    hw_ref[0] = hi | lo


def _fused_body(hexp_ref, pgt_ref, egt_ref,
                wj1_ref, bj1_ref, pw2_ref, pb2_ref, ew2_ref, eb2_ref,
                pwl_ref, pbl_ref, ewl_ref, ebl_ref,
                ppjw_ref, epjw_ref, pjb_ref,
                ha_ref, pp_ref, ep_ref):
    hw = hexp_ref[0]
    xlo = lax.bitcast_convert_type(lax.shift_left(hw, 16), jnp.float32)
    xhi = lax.bitcast_convert_type(hw & _HIMASK, jnp.float32)
    x = jnp.concatenate([xlo, xhi], axis=1)

    p = pgt_ref[0]
    e = egt_ref[0]
    ha_ref[0] = x + (p * ppjw_ref[...] + (e * epjw_ref[...] + pjb_ref[...]))
    h1 = jax.nn.relu(_conv3(x, wj1_ref, bj1_ref[...]))
    h2p = jax.nn.relu(_conv3(h1[:, :F], pw2_ref, pb2_ref[...]))
    h2e = jax.nn.relu(_conv3(h1[:, F:], ew2_ref, eb2_ref[...]))
    pp_ref[0] = jnp.dot(h2p, pwl_ref[...], preferred_element_type=jnp.float32) + pbl_ref[...]
    ep_ref[0] = jnp.dot(h2e, ewl_ref[...], preferred_element_type=jnp.float32) + ebl_ref[...]


def _full(bs):
    n = len(bs)
    return pl.BlockSpec(bs, lambda b: (b,) + (0,) * (n - 1))


def _rep(bs):
    n = len(bs)
    return pl.BlockSpec(bs, lambda b: (0,) * n)


def kernel(H, D_gt, P_gt, E_gt, dp_w1, dp_b1, dp_w2, dp_b2, dp_wl, dp_bl,
           pp_w1, pp_b1, pp_w2, pp_b2, pp_wl, pp_bl,
           ep_w1, ep_b1, ep_w2, ep_b2, ep_wl, ep_bl,
           ppj_w, ppj_b, epj_w, epj_b):
    f32 = jnp.float32
    bf16 = jnp.bfloat16
    def taps(w):
        return jnp.transpose(w, (2, 1, 0)).astype(bf16)
    dp_w1t, dp_w2t = taps(dp_w1), taps(dp_w2)
    wj1 = jnp.concatenate([taps(pp_w1), taps(ep_w1)], axis=2)
    bj1 = jnp.concatenate([pp_b1, ep_b1])[None, :]
    pw2t, ew2t = taps(pp_w2), taps(ep_w2)

    d_pred, h_packed = pl.pallas_call(
        _dp_pack_body,
        grid=(B,),
        in_specs=[_full((1, S, D_MODEL)), _rep((3, D_MODEL, F)), _rep((1, F)),
                  _rep((3, F, F)), _rep((1, F)), _rep((F, 1)), _rep((1, 1))],
        out_specs=[_full((1, S, 1)), _full((1, S, _HW))],
        out_shape=[jax.ShapeDtypeStruct((B, S, 1), f32),
                   jax.ShapeDtypeStruct((B, S, _HW), jnp.int32)],
    )(H, dp_w1t, dp_b1[None, :], dp_w2t, dp_b2[None, :], dp_wl, dp_bl[None, :])

    expand = pl.kernel(
        _expand_sc_body,
        out_type=jax.ShapeDtypeStruct((B * MAX_T, _HW), jnp.int32),
        mesh=plsc.VectorSubcoreMesh(core_axis_name="c", subcore_axis_name="s"),
        scratch_types=[
            pltpu.VMEM((S,), jnp.int32),
            pltpu.VMEM((S,), jnp.int32),
            pltpu.VMEM((_T_TILE,), jnp.int32),
            pltpu.VMEM((_T_TILE,), jnp.int32),
            pltpu.VMEM((3, _GCH, _HW), jnp.int32),
            pltpu.SemaphoreType.DMA,
            pltpu.SemaphoreType.DMA,
            pltpu.SemaphoreType.DMA,
        ],
        compiler_params=pltpu.CompilerParams(needs_layout_passes=False),
    )
    h_exp = expand(h_packed.reshape(B * S, _HW), D_gt).reshape(B, MAX_T, _HW)

    ha, ppred, epred = pl.pallas_call(
        _fused_body,
        grid=(B,),
        in_specs=[_full((1, MAX_T, _HW)),
                  _full((1, MAX_T, 1)), _full((1, MAX_T, 1)),
                  _rep((3, D_MODEL, 2 * F)), _rep((1, 2 * F)),
                  _rep((3, F, F)), _rep((1, F)), _rep((3, F, F)), _rep((1, F)),
                  _rep((F, 1)), _rep((1, 1)), _rep((F, 1)), _rep((1, 1)),
                  _rep((1, D_MODEL)), _rep((1, D_MODEL)), _rep((1, D_MODEL))],
        out_specs=[_full((1, MAX_T, D_MODEL)),
                   _full((1, MAX_T, 1)), _full((1, MAX_T, 1))],
        out_shape=[jax.ShapeDtypeStruct((B, MAX_T, D_MODEL), f32),
                   jax.ShapeDtypeStruct((B, MAX_T, 1), f32),
                   jax.ShapeDtypeStruct((B, MAX_T, 1), f32)],
    )(h_exp, P_gt[..., None], E_gt[..., None],
      wj1, bj1, pw2t, pp_b2[None, :], ew2t, ep_b2[None, :],
      pp_wl, pp_bl[None, :], ep_wl, ep_bl[None, :],
      ppj_w[None, :], epj_w[None, :], (ppj_b + epj_b)[None, :])

    return (ha, d_pred[..., 0], ppred[..., 0], epred[..., 0])


# speedup vs baseline: 1.0529x; 1.0382x over previous
import functools

import jax
import jax.numpy as jnp
from jax import lax
from jax.experimental import pallas as pl
from jax.experimental.pallas import tpu as pltpu
from jax.experimental.pallas import tpu_sc as plsc

B, S, D_MODEL = 16, 512, 256
MAX_T = 2048
F = 256
_HW = D_MODEL // 2

_T_TILE = MAX_T // 2
_GCH = 128


def _expand_sc_body(h_hbm, dgt_hbm, out_hbm,
                    d_v, c_v, cnt_v, idx_v, rows_v, sem0, sem1, sem2):
    i32 = jnp.int32
    wid = lax.axis_index("s") * 2 + lax.axis_index("c")
    b = wid // 2
    t0 = (wid % 2) * _T_TILE

    pltpu.sync_copy(dgt_hbm.at[b], d_v)

    carry = jnp.zeros((), i32)
    for j in range(S // 16):
        x = jnp.maximum(d_v[pl.ds(j * 16, 16)], 0)
        c_v[pl.ds(j * 16, 16)] = plsc.cumsum(x) + carry
        carry = carry + jnp.sum(x)
    total = carry

    zeros16 = jnp.zeros((16,), i32)
    for k in range(_T_TILE // 16):
        cnt_v[pl.ds(k * 16, 16)] = zeros16
    base = jnp.zeros((), i32)
    ones16 = jnp.full((16,), 1, i32)
    for j in range(S // 16):
        cj = c_v[pl.ds(j * 16, 16)]
        pos = cj - t0
        in_tile = (pos >= 0) & (pos < _T_TILE)
        plsc.addupdate_scatter(cnt_v, [pos], ones16, mask=in_tile)
        base = base + jnp.sum(jnp.where(cj < t0, 1, 0).astype(i32))

    run = base
    boff = b * S
    for k in range(_T_TILE // 16):
        cum = plsc.cumsum(cnt_v[pl.ds(k * 16, 16)]) + run
        run = run + jnp.sum(cnt_v[pl.ds(k * 16, 16)])
        idx_v[pl.ds(k * 16, 16)] = jnp.where(cum < S, cum + boff, boff)

    out_base = b * MAX_T + t0
    zf = jnp.zeros((16,), i32)

    def _zero_tail(buf, k):
        nv = jnp.clip(total - (t0 + k * _GCH), 0, _GCH)

        def body(r, _):
            for w in range(_HW // 16):
                buf[r, pl.ds(w * 16, 16)] = zf
            return 0

        lax.fori_loop(nv, _GCH, body, 0)

    sems = (sem0, sem1, sem2)
    nbuf = 3
    nch = _T_TILE // _GCH
    cps = [None] * nch

    def _drain(k):
        cps[k].wait()
        _zero_tail(rows_v.at[k % nbuf], k)
        pltpu.sync_copy(rows_v.at[k % nbuf],
                        out_hbm.at[pl.ds(out_base + k * _GCH, _GCH)])

    for k in range(nch):
        cps[k] = pltpu.async_copy(
            h_hbm.at[idx_v.at[pl.ds(k * _GCH, _GCH)]], rows_v.at[k % nbuf],
            sems[k % nbuf])
        if k >= nbuf - 1:
            _drain(k - (nbuf - 1))
    for k in range(nch - (nbuf - 1), nch):
        _drain(k)


def _conv3(x, w_ref, b):
    cin = x.shape[1]
    xc = x if x.dtype == jnp.bfloat16 else x.astype(jnp.bfloat16)
    zrow = jnp.zeros((1, cin), jnp.bfloat16)
    xm = jnp.concatenate([zrow, xc[:-1]], axis=0)
    xp = jnp.concatenate([xc[1:], zrow], axis=0)
    y = jnp.dot(xm, w_ref[0], preferred_element_type=jnp.float32)
    y = y + jnp.dot(xc, w_ref[1], preferred_element_type=jnp.float32)
    y = y + jnp.dot(xp, w_ref[2], preferred_element_type=jnp.float32)
    return y + b


_ROUND = 0x8000
_HIMASK = -65536


def _fused_body(hexp_ref, h_ref, pgt_ref, egt_ref,
                dw1_ref, db1_ref, dw2_ref, db2_ref, dwl_ref, dbl_ref,
                wj1_ref, bj1_ref, pw2_ref, pb2_ref, ew2_ref, eb2_ref,
                pwl_ref, pbl_ref, ewl_ref, ebl_ref,
                ppjw_ref, epjw_ref, pjb_ref,
                ha_ref, dp_ref, pp_ref, ep_ref):
    xd = h_ref[0]
    g1 = jax.nn.relu(_conv3(xd, dw1_ref, db1_ref[...]))
    g2 = jax.nn.relu(_conv3(g1, dw2_ref, db2_ref[...]))
    dp_ref[0] = jnp.dot(g2, dwl_ref[...], preferred_element_type=jnp.float32) + dbl_ref[...]

    hw = hexp_ref[0]
    xlo = lax.bitcast_convert_type(lax.shift_left(hw, 16), jnp.float32)
    xhi = lax.bitcast_convert_type(hw & _HIMASK, jnp.float32)
    x = jnp.concatenate([xlo, xhi], axis=1)

    p = pgt_ref[0]
    e = egt_ref[0]
    ha_ref[0] = x + (p * ppjw_ref[...] + (e * epjw_ref[...] + pjb_ref[...]))
    h1 = jax.nn.relu(_conv3(x, wj1_ref, bj1_ref[...]))
    h2p = jax.nn.relu(_conv3(h1[:, :F], pw2_ref, pb2_ref[...]))
    h2e = jax.nn.relu(_conv3(h1[:, F:], ew2_ref, eb2_ref[...]))
    pp_ref[0] = jnp.dot(h2p, pwl_ref[...], preferred_element_type=jnp.float32) + pbl_ref[...]
    ep_ref[0] = jnp.dot(h2e, ewl_ref[...], preferred_element_type=jnp.float32) + ebl_ref[...]


def _full(bs):
    n = len(bs)
    return pl.BlockSpec(bs, lambda b: (b,) + (0,) * (n - 1))


def _rep(bs):
    n = len(bs)
    return pl.BlockSpec(bs, lambda b: (0,) * n)


def kernel(H, D_gt, P_gt, E_gt, dp_w1, dp_b1, dp_w2, dp_b2, dp_wl, dp_bl,
           pp_w1, pp_b1, pp_w2, pp_b2, pp_wl, pp_bl,
           ep_w1, ep_b1, ep_w2, ep_b2, ep_wl, ep_bl,
           ppj_w, ppj_b, epj_w, epj_b):
    f32 = jnp.float32
    bf16 = jnp.bfloat16
    def taps(w):
        return jnp.transpose(w, (2, 1, 0)).astype(bf16)
    dp_w1t, dp_w2t = taps(dp_w1), taps(dp_w2)
    wj1 = jnp.concatenate([taps(pp_w1), taps(ep_w1)], axis=2)
    bj1 = jnp.concatenate([pp_b1, ep_b1])[None, :]
    pw2t, ew2t = taps(pp_w2), taps(ep_w2)

    bits = lax.bitcast_convert_type(H.reshape(B * S, D_MODEL), jnp.int32)
    h_packed = ((bits[:, _HW:] + _ROUND) & _HIMASK) | lax.shift_right_logical(
        bits[:, :_HW] + _ROUND, 16)

    expand = pl.kernel(
        _expand_sc_body,
        out_type=jax.ShapeDtypeStruct((B * MAX_T, _HW), jnp.int32),
        mesh=plsc.VectorSubcoreMesh(core_axis_name="c", subcore_axis_name="s"),
        scratch_types=[
            pltpu.VMEM((S,), jnp.int32),
            pltpu.VMEM((S,), jnp.int32),
            pltpu.VMEM((_T_TILE,), jnp.int32),
            pltpu.VMEM((_T_TILE,), jnp.int32),
            pltpu.VMEM((3, _GCH, _HW), jnp.int32),
            pltpu.SemaphoreType.DMA,
            pltpu.SemaphoreType.DMA,
            pltpu.SemaphoreType.DMA,
        ],
        compiler_params=pltpu.CompilerParams(needs_layout_passes=False),
    )
    h_exp = expand(h_packed, D_gt).reshape(B, MAX_T, _HW)

    ha, d_pred, ppred, epred = pl.pallas_call(
        _fused_body,
        grid=(B,),
        in_specs=[_full((1, MAX_T, _HW)), _full((1, S, D_MODEL)),
                  _full((1, MAX_T, 1)), _full((1, MAX_T, 1)),
                  _rep((3, D_MODEL, F)), _rep((1, F)),
                  _rep((3, F, F)), _rep((1, F)), _rep((F, 1)), _rep((1, 1)),
                  _rep((3, D_MODEL, 2 * F)), _rep((1, 2 * F)),
                  _rep((3, F, F)), _rep((1, F)), _rep((3, F, F)), _rep((1, F)),
                  _rep((F, 1)), _rep((1, 1)), _rep((F, 1)), _rep((1, 1)),
                  _rep((1, D_MODEL)), _rep((1, D_MODEL)), _rep((1, D_MODEL))],
        out_specs=[_full((1, MAX_T, D_MODEL)), _full((1, S, 1)),
                   _full((1, MAX_T, 1)), _full((1, MAX_T, 1))],
        out_shape=[jax.ShapeDtypeStruct((B, MAX_T, D_MODEL), f32),
                   jax.ShapeDtypeStruct((B, S, 1), f32),
                   jax.ShapeDtypeStruct((B, MAX_T, 1), f32),
                   jax.ShapeDtypeStruct((B, MAX_T, 1), f32)],
    )(h_exp, H, P_gt[..., None], E_gt[..., None],
      dp_w1t, dp_b1[None, :], dp_w2t, dp_b2[None, :], dp_wl, dp_bl[None, :],
      wj1, bj1, pw2t, pp_b2[None, :], ew2t, ep_b2[None, :],
      pp_wl, pp_bl[None, :], ep_wl, ep_bl[None, :],
      ppj_w[None, :], epj_w[None, :], (ppj_b + epj_b)[None, :])

    return (ha, d_pred[..., 0], ppred[..., 0], epred[..., 0])


# speedup vs baseline: 1.0553x; 1.0023x over previous
import functools

import jax
import jax.numpy as jnp
from jax import lax
from jax.experimental import pallas as pl
from jax.experimental.pallas import tpu as pltpu
from jax.experimental.pallas import tpu_sc as plsc

B, S, D_MODEL = 16, 512, 256
MAX_T = 2048
F = 256
_HW = D_MODEL // 2

_T_TILE = MAX_T
_GCH = 128


def _expand_sc_body(h_hbm, dgt_hbm, out_hbm,
                    d_v, c_v, cnt_v, idx_v, rows_v, sem0, sem1, sem2):
    i32 = jnp.int32
    b = lax.axis_index("s")
    t0 = 0

    pltpu.sync_copy(dgt_hbm.at[b], d_v)

    carry = jnp.zeros((), i32)
    for j in range(S // 16):
        x = jnp.maximum(d_v[pl.ds(j * 16, 16)], 0)
        c_v[pl.ds(j * 16, 16)] = plsc.cumsum(x) + carry
        carry = carry + jnp.sum(x)
    total = carry

    zeros16 = jnp.zeros((16,), i32)
    for k in range(_T_TILE // 16):
        cnt_v[pl.ds(k * 16, 16)] = zeros16
    base = jnp.zeros((), i32)
    ones16 = jnp.full((16,), 1, i32)
    for j in range(S // 16):
        cj = c_v[pl.ds(j * 16, 16)]
        pos = cj - t0
        in_tile = (pos >= 0) & (pos < _T_TILE)
        plsc.addupdate_scatter(cnt_v, [pos], ones16, mask=in_tile)
        base = base + jnp.sum(jnp.where(cj < t0, 1, 0).astype(i32))

    run = base
    boff = b * S
    for k in range(_T_TILE // 16):
        cum = plsc.cumsum(cnt_v[pl.ds(k * 16, 16)]) + run
        run = run + jnp.sum(cnt_v[pl.ds(k * 16, 16)])
        idx_v[pl.ds(k * 16, 16)] = jnp.where(cum < S, cum + boff, boff)

    out_base = b * MAX_T + t0
    zf = jnp.zeros((16,), i32)

    def _zero_tail(buf, k):
        nv = jnp.clip(total - (t0 + k * _GCH), 0, _GCH)

        def body(r, _):
            for w in range(_HW // 16):
                buf[r, pl.ds(w * 16, 16)] = zf
            return 0

        lax.fori_loop(nv, _GCH, body, 0)

    sems = (sem0, sem1, sem2)
    nbuf = 3
    nch = _T_TILE // _GCH
    cps = [None] * nch

    def _drain(k):
        cps[k].wait()
        _zero_tail(rows_v.at[k % nbuf], k)
        pltpu.sync_copy(rows_v.at[k % nbuf],
                        out_hbm.at[pl.ds(out_base + k * _GCH, _GCH)])

    for k in range(nch):
        cps[k] = pltpu.async_copy(
            h_hbm.at[idx_v.at[pl.ds(k * _GCH, _GCH)]], rows_v.at[k % nbuf],
            sems[k % nbuf])
        if k >= nbuf - 1:
            _drain(k - (nbuf - 1))
    for k in range(nch - (nbuf - 1), nch):
        _drain(k)


def _conv3(x, w_ref, b):
    cin = x.shape[1]
    xc = x if x.dtype == jnp.bfloat16 else x.astype(jnp.bfloat16)
    zrow = jnp.zeros((1, cin), jnp.bfloat16)
    xm = jnp.concatenate([zrow, xc[:-1]], axis=0)
    xp = jnp.concatenate([xc[1:], zrow], axis=0)
    y = jnp.dot(xm, w_ref[0], preferred_element_type=jnp.float32)
    y = y + jnp.dot(xc, w_ref[1], preferred_element_type=jnp.float32)
    y = y + jnp.dot(xp, w_ref[2], preferred_element_type=jnp.float32)
    return y + b


_ROUND = 0x8000
_HIMASK = -65536


def _fused_body(hexp_ref, h_ref, pgt_ref, egt_ref,
                dw1_ref, db1_ref, dw2_ref, db2_ref, dwl_ref, dbl_ref,
                wj1_ref, bj1_ref, pw2_ref, pb2_ref, ew2_ref, eb2_ref,
                pwl_ref, pbl_ref, ewl_ref, ebl_ref,
                ppjw_ref, epjw_ref, pjb_ref,
                ha_ref, dp_ref, pp_ref, ep_ref):
    xd = h_ref[0]
    g1 = jax.nn.relu(_conv3(xd, dw1_ref, db1_ref[...]))
    g2 = jax.nn.relu(_conv3(g1, dw2_ref, db2_ref[...]))
    dp_ref[0] = jnp.dot(g2, dwl_ref[...], preferred_element_type=jnp.float32) + dbl_ref[...]

    hw = hexp_ref[0]
    xlo = lax.bitcast_convert_type(lax.shift_left(hw, 16), jnp.float32)
    xhi = lax.bitcast_convert_type(hw & _HIMASK, jnp.float32)
    x = jnp.concatenate([xlo, xhi], axis=1)

    p = pgt_ref[0]
    e = egt_ref[0]
    ha_ref[0] = x + (p * ppjw_ref[...] + (e * epjw_ref[...] + pjb_ref[...]))
    h1 = jax.nn.relu(_conv3(x, wj1_ref, bj1_ref[...]))
    h2p = jax.nn.relu(_conv3(h1[:, :F], pw2_ref, pb2_ref[...]))
    h2e = jax.nn.relu(_conv3(h1[:, F:], ew2_ref, eb2_ref[...]))
    pp_ref[0] = jnp.dot(h2p, pwl_ref[...], preferred_element_type=jnp.float32) + pbl_ref[...]
    ep_ref[0] = jnp.dot(h2e, ewl_ref[...], preferred_element_type=jnp.float32) + ebl_ref[...]


def _full(bs):
    n = len(bs)
    return pl.BlockSpec(bs, lambda b: (b,) + (0,) * (n - 1))


def _rep(bs):
    n = len(bs)
    return pl.BlockSpec(bs, lambda b: (0,) * n)


def kernel(H, D_gt, P_gt, E_gt, dp_w1, dp_b1, dp_w2, dp_b2, dp_wl, dp_bl,
           pp_w1, pp_b1, pp_w2, pp_b2, pp_wl, pp_bl,
           ep_w1, ep_b1, ep_w2, ep_b2, ep_wl, ep_bl,
           ppj_w, ppj_b, epj_w, epj_b):
    f32 = jnp.float32
    bf16 = jnp.bfloat16
    def taps(w):
        return jnp.transpose(w, (2, 1, 0)).astype(bf16)
    dp_w1t, dp_w2t = taps(dp_w1), taps(dp_w2)
    wj1 = jnp.concatenate([taps(pp_w1), taps(ep_w1)], axis=2)
    bj1 = jnp.concatenate([pp_b1, ep_b1])[None, :]
    pw2t, ew2t = taps(pp_w2), taps(ep_w2)

    bits = lax.bitcast_convert_type(H.reshape(B * S, D_MODEL), jnp.int32)
    h_packed = ((bits[:, _HW:] + _ROUND) & _HIMASK) | lax.shift_right_logical(
        bits[:, :_HW] + _ROUND, 16)

    expand = pl.kernel(
        _expand_sc_body,
        out_type=jax.ShapeDtypeStruct((B * MAX_T, _HW), jnp.int32),
        mesh=plsc.VectorSubcoreMesh(core_axis_name="c", subcore_axis_name="s",
                                    num_cores=1),
        scratch_types=[
            pltpu.VMEM((S,), jnp.int32),
            pltpu.VMEM((S,), jnp.int32),
            pltpu.VMEM((_T_TILE,), jnp.int32),
            pltpu.VMEM((_T_TILE,), jnp.int32),
            pltpu.VMEM((3, _GCH, _HW), jnp.int32),
            pltpu.SemaphoreType.DMA,
            pltpu.SemaphoreType.DMA,
            pltpu.SemaphoreType.DMA,
        ],
        compiler_params=pltpu.CompilerParams(needs_layout_passes=False),
    )
    h_exp = expand(h_packed, D_gt).reshape(B, MAX_T, _HW)

    ha, d_pred, ppred, epred = pl.pallas_call(
        _fused_body,
        grid=(B,),
        in_specs=[_full((1, MAX_T, _HW)), _full((1, S, D_MODEL)),
                  _full((1, MAX_T, 1)), _full((1, MAX_T, 1)),
                  _rep((3, D_MODEL, F)), _rep((1, F)),
                  _rep((3, F, F)), _rep((1, F)), _rep((F, 1)), _rep((1, 1)),
                  _rep((3, D_MODEL, 2 * F)), _rep((1, 2 * F)),
                  _rep((3, F, F)), _rep((1, F)), _rep((3, F, F)), _rep((1, F)),
                  _rep((F, 1)), _rep((1, 1)), _rep((F, 1)), _rep((1, 1)),
                  _rep((1, D_MODEL)), _rep((1, D_MODEL)), _rep((1, D_MODEL))],
        out_specs=[_full((1, MAX_T, D_MODEL)), _full((1, S, 1)),
                   _full((1, MAX_T, 1)), _full((1, MAX_T, 1))],
        out_shape=[jax.ShapeDtypeStruct((B, MAX_T, D_MODEL), f32),
                   jax.ShapeDtypeStruct((B, S, 1), f32),
                   jax.ShapeDtypeStruct((B, MAX_T, 1), f32),
                   jax.ShapeDtypeStruct((B, MAX_T, 1), f32)],
    )(h_exp, H, P_gt[..., None], E_gt[..., None],
      dp_w1t, dp_b1[None, :], dp_w2t, dp_b2[None, :], dp_wl, dp_bl[None, :],
      wj1, bj1, pw2t, pp_b2[None, :], ew2t, ep_b2[None, :],
      pp_wl, pp_bl[None, :], ep_wl, ep_bl[None, :],
      ppj_w[None, :], epj_w[None, :], (ppj_b + epj_b)[None, :])

    return (ha, d_pred[..., 0], ppred[..., 0], epred[..., 0])


# speedup vs baseline: 1.0557x; 1.0003x over previous
import functools

import jax
import jax.numpy as jnp
from jax import lax
from jax.experimental import pallas as pl
from jax.experimental.pallas import tpu as pltpu
from jax.experimental.pallas import tpu_sc as plsc

B, S, D_MODEL = 16, 512, 256
MAX_T = 2048
F = 256
_HW = D_MODEL // 2

_T_TILE = MAX_T
_GCH = 128


def _expand_sc_body(h_hbm, dgt_hbm, out_hbm,
                    d_v, c_v, cnt_v, idx_v, rows_v, sem0, sem1, sem2):
    i32 = jnp.int32
    b = lax.axis_index("s")
    t0 = 0

    pltpu.sync_copy(dgt_hbm.at[b], d_v)

    carry = jnp.zeros((), i32)
    for j in range(S // 16):
        x = jnp.maximum(d_v[pl.ds(j * 16, 16)], 0)
        c_v[pl.ds(j * 16, 16)] = plsc.cumsum(x) + carry
        carry = carry + jnp.sum(x)
    total = carry

    zeros16 = jnp.zeros((16,), i32)
    for k in range(_T_TILE // 16):
        cnt_v[pl.ds(k * 16, 16)] = zeros16
    base = jnp.zeros((), i32)
    ones16 = jnp.full((16,), 1, i32)
    for j in range(S // 16):
        cj = c_v[pl.ds(j * 16, 16)]
        pos = cj - t0
        in_tile = (pos >= 0) & (pos < _T_TILE)
        plsc.addupdate_scatter(cnt_v, [pos], ones16, mask=in_tile)
        base = base + jnp.sum(jnp.where(cj < t0, 1, 0).astype(i32))

    run = base
    boff = b * S
    for k in range(_T_TILE // 16):
        cum = plsc.cumsum(cnt_v[pl.ds(k * 16, 16)]) + run
        run = run + jnp.sum(cnt_v[pl.ds(k * 16, 16)])
        idx_v[pl.ds(k * 16, 16)] = jnp.where(cum < S, cum + boff, boff)

    out_base = b * MAX_T + t0
    zf = jnp.zeros((16,), i32)

    def _zero_tail(buf, k):
        nv = jnp.clip(total - (t0 + k * _GCH), 0, _GCH)

        def body(r, _):
            for w in range(_HW // 16):
                buf[r, pl.ds(w * 16, 16)] = zf
            return 0

        lax.fori_loop(nv, _GCH, body, 0)

    sems = (sem0, sem1, sem2)
    nbuf = 3
    nch = _T_TILE // _GCH
    cps = [None] * nch

    def _drain(k):
        cps[k].wait()
        _zero_tail(rows_v.at[k % nbuf], k)
        pltpu.sync_copy(rows_v.at[k % nbuf],
                        out_hbm.at[pl.ds(out_base + k * _GCH, _GCH)])

    for k in range(nch):
        cps[k] = pltpu.async_copy(
            h_hbm.at[idx_v.at[pl.ds(k * _GCH, _GCH)]], rows_v.at[k % nbuf],
            sems[k % nbuf])
        if k >= nbuf - 1:
            _drain(k - (nbuf - 1))
    for k in range(nch - (nbuf - 1), nch):
        _drain(k)


def _conv3(x, w_ref, b):
    cin = x.shape[1]
    xc = x if x.dtype == jnp.bfloat16 else x.astype(jnp.bfloat16)
    zrow = jnp.zeros((1, cin), jnp.bfloat16)
    xm = jnp.concatenate([zrow, xc[:-1]], axis=0)
    xp = jnp.concatenate([xc[1:], zrow], axis=0)
    y = jnp.dot(xm, w_ref[0], preferred_element_type=jnp.float32)
    y = y + jnp.dot(xc, w_ref[1], preferred_element_type=jnp.float32)
    y = y + jnp.dot(xp, w_ref[2], preferred_element_type=jnp.float32)
    return y + b


_ROUND = 0x8000
_HIMASK = -65536


def _fused_body(hexp_ref, h_ref, pgt_ref, egt_ref,
                dw1_ref, db1_ref, dw2_ref, db2_ref, dwl_ref, dbl_ref,
                wj1_ref, bj1_ref, pw2_ref, pb2_ref, ew2_ref, eb2_ref,
                pwl_ref, pbl_ref, ewl_ref, ebl_ref,
                ppjw_ref, epjw_ref, pjb_ref,
                ha_ref, dp_ref, pp_ref, ep_ref):
    xd = h_ref[0]
    g1 = jax.nn.relu(_conv3(xd, dw1_ref, db1_ref[...]))
    g2 = jax.nn.relu(_conv3(g1, dw2_ref, db2_ref[...]))
    dp_ref[0] = jnp.dot(g2, dwl_ref[...], preferred_element_type=jnp.float32) + dbl_ref[...]

    hw = hexp_ref[0]
    xlo = lax.bitcast_convert_type(lax.shift_left(hw, 16), jnp.float32)
    xhi = lax.bitcast_convert_type(hw & _HIMASK, jnp.float32)
    x = jnp.concatenate([xlo, xhi], axis=1)

    p = pgt_ref[0]
    e = egt_ref[0]
    ha_ref[0] = x + (p * ppjw_ref[...] + (e * epjw_ref[...] + pjb_ref[...]))
    h1 = jax.nn.relu(_conv3(x, wj1_ref, bj1_ref[...]))
    h2p = jax.nn.relu(_conv3(h1[:, :F], pw2_ref, pb2_ref[...]))
    h2e = jax.nn.relu(_conv3(h1[:, F:], ew2_ref, eb2_ref[...]))
    pp_ref[0] = jnp.dot(h2p, pwl_ref[...], preferred_element_type=jnp.float32) + pbl_ref[...]
    ep_ref[0] = jnp.dot(h2e, ewl_ref[...], preferred_element_type=jnp.float32) + ebl_ref[...]


def _full(bs):
    n = len(bs)
    return pl.BlockSpec(bs, lambda b: (b,) + (0,) * (n - 1))


def _rep(bs):
    n = len(bs)
    return pl.BlockSpec(bs, lambda b: (0,) * n)


def kernel(H, D_gt, P_gt, E_gt, dp_w1, dp_b1, dp_w2, dp_b2, dp_wl, dp_bl,
           pp_w1, pp_b1, pp_w2, pp_b2, pp_wl, pp_bl,
           ep_w1, ep_b1, ep_w2, ep_b2, ep_wl, ep_bl,
           ppj_w, ppj_b, epj_w, epj_b):
    f32 = jnp.float32
    bf16 = jnp.bfloat16
    def taps(w):
        return jnp.transpose(w, (2, 1, 0)).astype(bf16)
    dp_w1t, dp_w2t = taps(dp_w1), taps(dp_w2)
    wj1 = jnp.concatenate([taps(pp_w1), taps(ep_w1)], axis=2)
    bj1 = jnp.concatenate([pp_b1, ep_b1])[None, :]
    pw2t, ew2t = taps(pp_w2), taps(ep_w2)

    bits = lax.bitcast_convert_type(H.reshape(B * S, D_MODEL), jnp.int32)
    h_packed = ((bits[:, _HW:] + _ROUND) & _HIMASK) | lax.shift_right_logical(
        bits[:, :_HW] + _ROUND, 16)

    expand = pl.kernel(
        _expand_sc_body,
        out_type=jax.ShapeDtypeStruct((B * MAX_T, _HW), jnp.int32),
        mesh=plsc.VectorSubcoreMesh(core_axis_name="c", subcore_axis_name="s",
                                    num_cores=1),
        scratch_types=[
            pltpu.VMEM((S,), jnp.int32),
            pltpu.VMEM((S,), jnp.int32),
            pltpu.VMEM((_T_TILE,), jnp.int32),
            pltpu.VMEM((_T_TILE,), jnp.int32),
            pltpu.VMEM((3, _GCH, _HW), jnp.int32),
            pltpu.SemaphoreType.DMA,
            pltpu.SemaphoreType.DMA,
            pltpu.SemaphoreType.DMA,
        ],
        compiler_params=pltpu.CompilerParams(needs_layout_passes=False),
    )
    h_exp = expand(h_packed, D_gt).reshape(B, MAX_T, _HW)

    ha, d_pred, ppred, epred = pl.pallas_call(
        _fused_body,
        grid=(B,),
        in_specs=[_full((1, MAX_T, _HW)), _full((1, S, D_MODEL)),
                  _full((1, MAX_T, 1)), _full((1, MAX_T, 1)),
                  _rep((3, D_MODEL, F)), _rep((1, F)),
                  _rep((3, F, F)), _rep((1, F)), _rep((F, 1)), _rep((1, 1)),
                  _rep((3, D_MODEL, 2 * F)), _rep((1, 2 * F)),
                  _rep((3, F, F)), _rep((1, F)), _rep((3, F, F)), _rep((1, F)),
                  _rep((F, 1)), _rep((1, 1)), _rep((F, 1)), _rep((1, 1)),
                  _rep((1, D_MODEL)), _rep((1, D_MODEL)), _rep((1, D_MODEL))],
        out_specs=[_full((1, MAX_T, D_MODEL)), _full((1, S, 1)),
                   _full((1, MAX_T, 1)), _full((1, MAX_T, 1))],
        out_shape=[jax.ShapeDtypeStruct((B, MAX_T, D_MODEL), f32),
                   jax.ShapeDtypeStruct((B, S, 1), f32),
                   jax.ShapeDtypeStruct((B, MAX_T, 1), f32),
                   jax.ShapeDtypeStruct((B, MAX_T, 1), f32)],
        compiler_params=pltpu.CompilerParams(
            dimension_semantics=("parallel",)),
    )(h_exp, H, P_gt[..., None], E_gt[..., None],
      dp_w1t, dp_b1[None, :], dp_w2t, dp_b2[None, :], dp_wl, dp_bl[None, :],
      wj1, bj1, pw2t, pp_b2[None, :], ew2t, ep_b2[None, :],
      pp_wl, pp_bl[None, :], ep_wl, ep_bl[None, :],
      ppj_w[None, :], epj_w[None, :], (ppj_b + epj_b)[None, :])

    return (ha, d_pred[..., 0], ppred[..., 0], epred[..., 0])
